# Initial kernel scaffold; baseline (speedup 1.0000x reference)
import functools

import jax
import jax.numpy as jnp
from jax import lax
from jax.experimental import pallas as pl
from jax.experimental.pallas import tpu as pltpu
from jax.experimental.pallas import tpu_sc as plsc

LANES = 16
CHUNK = 64
STREAM = 128


def kernel(weights, table_offsets, sparse_indices, sparse_offsets):
    n_bags = sparse_offsets.shape[0] - 1
    n_tables = table_offsets.shape[0]
    batch = n_bags // n_tables
    pool = sparse_indices.shape[0] // n_bags
    d = weights.shape[1]

    info = plsc.get_sparse_core_info()
    num_cores = info.num_cores
    nw = num_cores * info.num_subcores

    n_chunks = n_bags // CHUNK
    chunks_per_table = batch // CHUNK
    chunks_per_worker = n_chunks // nw
    idx_per_chunk = CHUNK * pool
    n_streams = idx_per_chunk // STREAM

    mesh = plsc.VectorSubcoreMesh(core_axis_name="c", subcore_axis_name="s")

    @functools.partial(
        pl.kernel,
        mesh=mesh,
        out_type=jax.ShapeDtypeStruct((n_bags, d), jnp.float32),
        scratch_types=[
            pltpu.VMEM((2 * LANES,), jnp.int32),
            pltpu.VMEM((idx_per_chunk,), jnp.int32),
            pltpu.VMEM((idx_per_chunk,), jnp.int32),
            pltpu.VMEM((idx_per_chunk, d), jnp.float32),
            pltpu.VMEM((CHUNK, d), jnp.float32),
            pltpu.SemaphoreType.DMA,
        ],
    )
    def body(w_hbm, toff_hbm, sidx_hbm, out_hbm,
             toff_v, lidx_v, gidx_v, rows_v, outc_v, sem):
        wid = lax.axis_index("s") * num_cores + lax.axis_index("c")
        pltpu.sync_copy(toff_hbm, toff_v.at[pl.ds(0, n_tables)])

        def chunk_body(i, carry):
            chunk = wid * chunks_per_worker + i
            table = chunk // chunks_per_table
            bag0 = chunk * CHUNK

            pltpu.sync_copy(sidx_hbm.at[pl.ds(bag0 * pool, idx_per_chunk)],
                            lidx_v)
            off = plsc.load_gather(
                toff_v, [jnp.full((LANES,), table, jnp.int32)])

            def adjust(v, c):
                sl = pl.ds(v * LANES, LANES)
                gidx_v[sl] = lidx_v[sl] + off
                return c
            lax.fori_loop(0, idx_per_chunk // LANES, adjust, 0)

            copies = [
                pltpu.async_copy(
                    w_hbm.at[gidx_v.at[pl.ds(j * STREAM, STREAM)]],
                    rows_v.at[pl.ds(j * STREAM, STREAM)],
                    sem,
                )
                for j in range(n_streams)
            ]
            for cp in copies:
                cp.wait()

            def bag(b, c):
                base = b * pool
                acc = rows_v[base, :]
                for j in range(1, pool):
                    acc = acc + rows_v[base + j, :]
                outc_v[b, :] = acc
                return c
            lax.fori_loop(0, CHUNK, bag, 0)

            pltpu.sync_copy(outc_v, out_hbm.at[pl.ds(bag0, CHUNK)])
            return carry

        lax.fori_loop(0, chunks_per_worker, chunk_body, 0)

    return body(weights, table_offsets, sparse_indices)


# speedup vs baseline: 3.7865x; 3.7865x over previous
import functools

import jax
import jax.numpy as jnp
from jax import lax
from jax.experimental import pallas as pl
from jax.experimental.pallas import tpu as pltpu
from jax.experimental.pallas import tpu_sc as plsc

LANES = 16
CHUNK = 64
STREAM = 128


def kernel(weights, table_offsets, sparse_indices, sparse_offsets):
    n_bags = sparse_offsets.shape[0] - 1
    n_tables = table_offsets.shape[0]
    batch = n_bags // n_tables
    pool = sparse_indices.shape[0] // n_bags
    d = weights.shape[1]
    rows_per_table = weights.shape[0] // n_tables

    info = plsc.get_sparse_core_info()
    num_cores = info.num_cores
    nw = num_cores * info.num_subcores

    n_chunks = n_bags // CHUNK
    chunks_per_table = batch // CHUNK
    chunks_per_worker = n_chunks // nw
    idx_per_chunk = CHUNK * pool
    n_streams = idx_per_chunk // STREAM

    mesh = plsc.VectorSubcoreMesh(core_axis_name="c", subcore_axis_name="s")

    @functools.partial(
        pl.kernel,
        mesh=mesh,
        out_type=jax.ShapeDtypeStruct((n_bags, d), jnp.float32),
        scratch_types=[
            pltpu.VMEM((idx_per_chunk,), jnp.int32),
            pltpu.VMEM((idx_per_chunk,), jnp.int32),
            pltpu.VMEM((idx_per_chunk, d), jnp.float32),
            pltpu.VMEM((CHUNK, d), jnp.float32),
            pltpu.SemaphoreType.DMA,
        ],
        compiler_params=pltpu.CompilerParams(use_tc_tiling_on_sc=False),
    )
    def body(w_hbm, sidx_hbm, out_hbm,
             lidx_v, gidx_v, rows_v, outc_v, sem):
        wid = lax.axis_index("s") * num_cores + lax.axis_index("c")

        def chunk_body(i, carry):
            chunk = wid * chunks_per_worker + i
            table = chunk // chunks_per_table
            bag0 = chunk * CHUNK

            pltpu.sync_copy(sidx_hbm.at[pl.ds(bag0 * pool, idx_per_chunk)],
                            lidx_v)
            off = jnp.full((LANES,), table * rows_per_table, jnp.int32)

            def adjust(v, c):
                sl = pl.ds(v * LANES, LANES)
                gidx_v[sl] = lidx_v[sl] + off
                return c
            lax.fori_loop(0, idx_per_chunk // LANES, adjust, 0)

            copies = [
                pltpu.async_copy(
                    w_hbm.at[gidx_v.at[pl.ds(j * STREAM, STREAM)]],
                    rows_v.at[pl.ds(j * STREAM, STREAM)],
                    sem,
                )
                for j in range(n_streams)
            ]
            for cp in copies:
                cp.wait()

            def bag(b, c):
                base = b * pool
                acc = rows_v[base, :]
                for j in range(1, pool):
                    acc = acc + rows_v[base + j, :]
                outc_v[b, :] = acc
                return c
            lax.fori_loop(0, CHUNK, bag, 0)

            pltpu.sync_copy(outc_v, out_hbm.at[pl.ds(bag0, CHUNK)])
            return carry

        lax.fori_loop(0, chunks_per_worker, chunk_body, 0)

    return body(weights, sparse_indices)


# speedup vs baseline: 4.1833x; 1.1048x over previous
import functools

import jax
import jax.numpy as jnp
from jax import lax
from jax.experimental import pallas as pl
from jax.experimental.pallas import tpu as pltpu
from jax.experimental.pallas import tpu_sc as plsc

LANES = 16
CHUNK = 64
STREAM = 128


def kernel(weights, table_offsets, sparse_indices, sparse_offsets):
    n_bags = sparse_offsets.shape[0] - 1
    n_tables = table_offsets.shape[0]
    batch = n_bags // n_tables
    pool = sparse_indices.shape[0] // n_bags
    d = weights.shape[1]
    rows_per_table = weights.shape[0] // n_tables

    info = plsc.get_sparse_core_info()
    num_cores = info.num_cores
    nw = num_cores * info.num_subcores

    n_chunks = n_bags // CHUNK
    chunks_per_table = batch // CHUNK
    n = n_chunks // nw
    pairs = n // 2
    idx_per_chunk = CHUNK * pool
    n_streams = idx_per_chunk // STREAM

    mesh = plsc.VectorSubcoreMesh(core_axis_name="c", subcore_axis_name="s")

    @functools.partial(
        pl.kernel,
        mesh=mesh,
        out_type=jax.ShapeDtypeStruct((n_bags, d), jnp.float32),
        scratch_types=[
            pltpu.VMEM((idx_per_chunk,), jnp.int32),
            pltpu.VMEM((idx_per_chunk,), jnp.int32),
            pltpu.VMEM((idx_per_chunk,), jnp.int32),
            pltpu.VMEM((idx_per_chunk,), jnp.int32),
            pltpu.VMEM((idx_per_chunk, d), jnp.float32),
            pltpu.VMEM((idx_per_chunk, d), jnp.float32),
            pltpu.VMEM((CHUNK, d), jnp.float32),
            pltpu.VMEM((CHUNK, d), jnp.float32),
            pltpu.SemaphoreType.DMA,
            pltpu.SemaphoreType.DMA,
            pltpu.SemaphoreType.DMA,
            pltpu.SemaphoreType.DMA,
            pltpu.SemaphoreType.DMA,
            pltpu.SemaphoreType.DMA,
        ],
        compiler_params=pltpu.CompilerParams(use_tc_tiling_on_sc=False),
    )
    def body(w_hbm, sidx_hbm, out_hbm,
             lidx0, lidx1, gidx0, gidx1, rows0, rows1, outc0, outc1,
             si0, si1, sg0, sg1, so0, so1):
        lidx = (lidx0, lidx1)
        gidx = (gidx0, gidx1)
        rows = (rows0, rows1)
        outc = (outc0, outc1)
        sem_i = (si0, si1)
        sem_g = (sg0, sg1)
        sem_o = (so0, so1)

        wid = lax.axis_index("s") * num_cores + lax.axis_index("c")
        c0 = wid * n

        def idx_desc(c, s):
            return pltpu.make_async_copy(
                sidx_hbm.at[pl.ds((c0 + c) * idx_per_chunk, idx_per_chunk)],
                lidx[s], sem_i[s])

        def gather_descs(s):
            return [
                pltpu.make_async_copy(
                    w_hbm.at[gidx[s].at[pl.ds(j * STREAM, STREAM)]],
                    rows[s].at[pl.ds(j * STREAM, STREAM)],
                    sem_g[s])
                for j in range(n_streams)
            ]

        def out_desc(c, s):
            return pltpu.make_async_copy(
                outc[s], out_hbm.at[pl.ds((c0 + c) * CHUNK, CHUNK)], sem_o[s])

        def adjust(c, s):
            table = (c0 + c) // chunks_per_table
            off = jnp.full((LANES,), 0, jnp.int32) + table * rows_per_table

            def f(v, carry):
                sl = pl.ds(v * LANES, LANES)
                gidx[s][sl] = lidx[s][sl] + off
                return carry
            lax.fori_loop(0, idx_per_chunk // LANES, f, 0)

        def accumulate(s):
            def f(b, carry):
                base = b * pool
                a = rows[s][base, :]
                for j in range(1, pool):
                    a = a + rows[s][base + j, :]
                outc[s][b, :] = a
                return carry
            lax.fori_loop(0, CHUNK, f, 0)

        def front(c, s, g):
            idx_desc(c, s).wait()
            adjust(c, s)
            for dsc in gather_descs(s):
                dsc.start()

            @pl.when(g < pairs - 1)
            def _():
                idx_desc(c + 2, s).start()

        def back(c, s):
            @pl.when(c >= 2)
            def _():
                out_desc(c - 2, s).wait()
            for dsc in gather_descs(s):
                dsc.wait()
            accumulate(s)
            out_desc(c, s).start()

        idx_desc(0, 0).start()
        idx_desc(1, 1).start()

        def pair_body(g, carry):
            c = 2 * g
            front(c, 0, g)

            @pl.when(g > 0)
            def _():
                back(c - 1, 1)
            front(c + 1, 1, g)
            back(c, 0)
            return carry

        lax.fori_loop(0, pairs, pair_body, 0)
        back(n - 1, 1)
        out_desc(n - 2, 0).wait()
        out_desc(n - 1, 1).wait()

    return body(weights, sparse_indices)


# speedup vs baseline: 4.6923x; 1.1217x over previous
import functools

import jax
import jax.numpy as jnp
from jax import lax
from jax.experimental import pallas as pl
from jax.experimental.pallas import tpu as pltpu
from jax.experimental.pallas import tpu_sc as plsc

LANES = 16
CHUNK = 64
STREAM = 128
W = 1024


def _make_conv(n_rows, d, num_cores, nw):
    n_full = n_rows // W
    tail = n_rows - n_full * W
    iters = (n_full + nw - 1) // nw
    pairs = iters // 2
    mesh = plsc.VectorSubcoreMesh(core_axis_name="c", subcore_axis_name="s")

    @functools.partial(
        pl.kernel,
        mesh=mesh,
        out_type=jax.ShapeDtypeStruct((n_rows * d,), jnp.float32),
        scratch_types=[
            pltpu.VMEM((d, W), jnp.float32),
            pltpu.VMEM((d, W), jnp.float32),
            pltpu.VMEM((W * d,), jnp.float32),
            pltpu.VMEM((W * d,), jnp.float32),
            pltpu.VMEM((tail * d,), jnp.float32),
            pltpu.SemaphoreType.DMA,
            pltpu.SemaphoreType.DMA,
            pltpu.SemaphoreType.DMA,
            pltpu.SemaphoreType.DMA,
        ],
        compiler_params=pltpu.CompilerParams(
            use_tc_tiling_on_sc=True, needs_layout_passes=False),
    )
    def conv(wt_hbm, tail_hbm, out_hbm,
             in0, in1, ot0, ot1, tail_v, si0, si1, so0, so1):
        in_v = (in0, in1)
        out_v = (ot0, ot1)
        sem_i = (si0, si1)
        sem_o = (so0, so1)
        wid = lax.axis_index("s") * num_cores + lax.axis_index("c")
        rows16 = lax.iota(jnp.int32, LANES)

        def in_desc(i, s):
            return pltpu.make_async_copy(
                wt_hbm.at[:, pl.ds((i * nw + wid) * W, W)], in_v[s], sem_i[s])

        def out_desc(i, s):
            return pltpu.make_async_copy(
                out_v[s], out_hbm.at[pl.ds((i * nw + wid) * W * d, W * d)],
                sem_o[s])

        def valid(i):
            return i * nw + wid < n_full

        def transpose(s):
            def f(v, carry):
                for u in range(8):
                    c = v * 8 + u
                    x = plsc.load_gather(
                        in_v[s], [rows16, jnp.full((LANES,), c, jnp.int32)])
                    out_v[s][pl.ds(c * d, d)] = x
                return carry
            lax.fori_loop(0, W // 8, f, 0)

        in_desc(0, 0).start()
        in_desc(1, 1).start()

        def step(i, s):
            @pl.when(valid(i))
            def _():
                @pl.when(i >= 2)
                def _():
                    out_desc(i - 2, s).wait()
                in_desc(i, s).wait()
                transpose(s)
                out_desc(i, s).start()

            @pl.when(valid(i + 2))
            def _():
                in_desc(i + 2, s).start()

        def pair_body(g, carry):
            step(2 * g, 0)
            step(2 * g + 1, 1)
            return carry

        lax.fori_loop(0, pairs, pair_body, 0)
        out_desc(0, 0).wait()
        out_desc(1, 1).wait()

        @pl.when(wid == nw - 1)
        def _():
            pltpu.sync_copy(tail_hbm, tail_v)
            pltpu.sync_copy(tail_v,
                            out_hbm.at[pl.ds(n_full * W * d, tail * d)])

    return conv


def _make_bag(n_rows, d, n_bags, n_tables, batch, pool, num_cores, nw):
    rows_per_table = n_rows // n_tables
    n_chunks = n_bags // CHUNK
    chunks_per_table = batch // CHUNK
    n = n_chunks // nw
    pairs = n // 2
    idx_per_chunk = CHUNK * pool
    n_streams = idx_per_chunk // STREAM
    mesh = plsc.VectorSubcoreMesh(core_axis_name="c", subcore_axis_name="s")

    @functools.partial(
        pl.kernel,
        mesh=mesh,
        out_type=jax.ShapeDtypeStruct((n_bags, d), jnp.float32),
        scratch_types=[
            pltpu.VMEM((idx_per_chunk,), jnp.int32),
            pltpu.VMEM((idx_per_chunk,), jnp.int32),
            pltpu.VMEM((idx_per_chunk,), jnp.int32),
            pltpu.VMEM((idx_per_chunk,), jnp.int32),
            pltpu.VMEM((idx_per_chunk, d), jnp.float32),
            pltpu.VMEM((idx_per_chunk, d), jnp.float32),
            pltpu.VMEM((CHUNK, d), jnp.float32),
            pltpu.VMEM((CHUNK, d), jnp.float32),
            pltpu.SemaphoreType.DMA,
            pltpu.SemaphoreType.DMA,
            pltpu.SemaphoreType.DMA,
            pltpu.SemaphoreType.DMA,
            pltpu.SemaphoreType.DMA,
            pltpu.SemaphoreType.DMA,
        ],
        compiler_params=pltpu.CompilerParams(use_tc_tiling_on_sc=False),
    )
    def body(w_hbm, sidx_hbm, out_hbm,
             lidx0, lidx1, gidx0, gidx1, rows0, rows1, outc0, outc1,
             si0, si1, sg0, sg1, so0, so1):
        lidx = (lidx0, lidx1)
        gidx = (gidx0, gidx1)
        rows = (rows0, rows1)
        outc = (outc0, outc1)
        sem_i = (si0, si1)
        sem_g = (sg0, sg1)
        sem_o = (so0, so1)

        wid = lax.axis_index("s") * num_cores + lax.axis_index("c")
        c0 = wid * n

        def idx_desc(c, s):
            return pltpu.make_async_copy(
                sidx_hbm.at[pl.ds((c0 + c) * idx_per_chunk, idx_per_chunk)],
                lidx[s], sem_i[s])

        def gather_descs(s):
            return [
                pltpu.make_async_copy(
                    w_hbm.at[gidx[s].at[pl.ds(j * STREAM, STREAM)]],
                    rows[s].at[pl.ds(j * STREAM, STREAM)],
                    sem_g[s])
                for j in range(n_streams)
            ]

        def out_desc(c, s):
            return pltpu.make_async_copy(
                outc[s], out_hbm.at[pl.ds((c0 + c) * CHUNK, CHUNK)], sem_o[s])

        def adjust(c, s):
            table = (c0 + c) // chunks_per_table
            off = jnp.full((LANES,), 0, jnp.int32) + table * rows_per_table

            def f(v, carry):
                sl = pl.ds(v * LANES, LANES)
                gidx[s][sl] = lidx[s][sl] + off
                return carry
            lax.fori_loop(0, idx_per_chunk // LANES, f, 0)

        def accumulate(s):
            def f(b, carry):
                base = b * pool
                a = rows[s][base, :]
                for j in range(1, pool):
                    a = a + rows[s][base + j, :]
                outc[s][b, :] = a
                return carry
            lax.fori_loop(0, CHUNK, f, 0)

        def front(c, s, g):
            idx_desc(c, s).wait()
            adjust(c, s)
            for dsc in gather_descs(s):
                dsc.start()

            @pl.when(g < pairs - 1)
            def _():
                idx_desc(c + 2, s).start()

        def back(c, s):
            @pl.when(c >= 2)
            def _():
                out_desc(c - 2, s).wait()
            for dsc in gather_descs(s):
                dsc.wait()
            accumulate(s)
            out_desc(c, s).start()

        idx_desc(0, 0).start()
        idx_desc(1, 1).start()

        def pair_body(g, carry):
            c = 2 * g
            front(c, 0, g)

            @pl.when(g > 0)
            def _():
                back(c - 1, 1)
            front(c + 1, 1, g)
            back(c, 0)
            return carry

        lax.fori_loop(0, pairs, pair_body, 0)
        back(n - 1, 1)
        out_desc(n - 2, 0).wait()
        out_desc(n - 1, 1).wait()

    return body


def kernel(weights, table_offsets, sparse_indices, sparse_offsets):
    n_bags = sparse_offsets.shape[0] - 1
    n_tables = table_offsets.shape[0]
    batch = n_bags // n_tables
    pool = sparse_indices.shape[0] // n_bags
    n_rows, d = weights.shape

    info = plsc.get_sparse_core_info()
    num_cores = info.num_cores
    nw = num_cores * info.num_subcores

    n_full = n_rows // W
    tail_arr = weights[n_full * W:, :].reshape((n_rows - n_full * W) * d)

    conv = _make_conv(n_rows, d, num_cores, nw)
    wlin = conv(weights.T, tail_arr)
    w2 = wlin.reshape(n_rows, d)

    bag = _make_bag(n_rows, d, n_bags, n_tables, batch, pool, num_cores, nw)
    return bag(w2, sparse_indices)


# speedup vs baseline: 16.1691x; 3.4459x over previous
import functools

import jax
import jax.numpy as jnp
from jax import lax
from jax.experimental import pallas as pl
from jax.experimental.pallas import tpu as pltpu
from jax.experimental.pallas import tpu_sc as plsc

LANES = 16
CHUNK = 128
STREAM = 128
W = 1024


def _make_conv(n_rows, d, num_cores, nw):
    n_full = n_rows // W
    tail = n_rows - n_full * W
    iters = (n_full + nw - 1) // nw
    pairs = iters // 2
    mesh = plsc.VectorSubcoreMesh(core_axis_name="c", subcore_axis_name="s")

    @functools.partial(
        pl.kernel,
        mesh=mesh,
        out_type=jax.ShapeDtypeStruct((n_rows * d,), jnp.float32),
        scratch_types=[
            pltpu.VMEM((d, W), jnp.float32),
            pltpu.VMEM((d, W), jnp.float32),
            pltpu.VMEM((W * d,), jnp.float32),
            pltpu.VMEM((W * d,), jnp.float32),
            pltpu.VMEM((tail * d,), jnp.float32),
            pltpu.SemaphoreType.DMA,
            pltpu.SemaphoreType.DMA,
            pltpu.SemaphoreType.DMA,
            pltpu.SemaphoreType.DMA,
        ],
        compiler_params=pltpu.CompilerParams(
            use_tc_tiling_on_sc=True, needs_layout_passes=False),
    )
    def conv(wt_hbm, tail_hbm, out_hbm,
             in0, in1, ot0, ot1, tail_v, si0, si1, so0, so1):
        in_v = (in0, in1)
        out_v = (ot0, ot1)
        sem_i = (si0, si1)
        sem_o = (so0, so1)
        wid = lax.axis_index("s") * num_cores + lax.axis_index("c")
        lane_stride = lax.iota(jnp.int32, LANES) * d

        def in_desc(i, s):
            return pltpu.make_async_copy(
                wt_hbm.at[:, pl.ds((i * nw + wid) * W, W)], in_v[s], sem_i[s])

        def out_desc(i, s):
            return pltpu.make_async_copy(
                out_v[s], out_hbm.at[pl.ds((i * nw + wid) * W * d, W * d)],
                sem_o[s])

        def valid(i):
            return i * nw + wid < n_full

        def transpose(s):
            @plsc.parallel_loop(0, W // LANES, unroll=4)
            def _(cb):
                for i in range(d):
                    x = in_v[s][i, pl.ds(cb * LANES, LANES)]
                    plsc.store_scatter(
                        out_v[s], [lane_stride + (cb * LANES * d + i)], x)

        in_desc(0, 0).start()
        in_desc(1, 1).start()

        def step(i, s):
            @pl.when(valid(i))
            def _():
                @pl.when(i >= 2)
                def _():
                    out_desc(i - 2, s).wait()
                in_desc(i, s).wait()
                transpose(s)
                out_desc(i, s).start()

            @pl.when(valid(i + 2))
            def _():
                in_desc(i + 2, s).start()

        def pair_body(g, carry):
            step(2 * g, 0)
            step(2 * g + 1, 1)
            return carry

        lax.fori_loop(0, pairs, pair_body, 0)
        out_desc(0, 0).wait()
        out_desc(1, 1).wait()

        @pl.when(wid == nw - 1)
        def _():
            pltpu.sync_copy(tail_hbm, tail_v)
            pltpu.sync_copy(tail_v,
                            out_hbm.at[pl.ds(n_full * W * d, tail * d)])

    return conv


def _make_bag(n_rows, d, n_bags, n_tables, batch, pool, num_cores, nw):
    rows_per_table = n_rows // n_tables
    n_chunks = n_bags // CHUNK
    chunks_per_table = batch // CHUNK
    n = n_chunks // nw
    pairs = n // 2
    idx_per_chunk = CHUNK * pool
    n_streams = idx_per_chunk // STREAM
    mesh = plsc.VectorSubcoreMesh(core_axis_name="c", subcore_axis_name="s")

    tile_elems = 8 * STREAM

    @functools.partial(
        pl.kernel,
        mesh=mesh,
        out_type=jax.ShapeDtypeStruct((n_bags * d,), jnp.float32),
        scratch_types=[
            pltpu.VMEM((idx_per_chunk,), jnp.int32),
            pltpu.VMEM((idx_per_chunk,), jnp.int32),
            pltpu.VMEM((idx_per_chunk,), jnp.int32),
            pltpu.VMEM((idx_per_chunk,), jnp.int32),
            pltpu.VMEM((idx_per_chunk, d), jnp.float32),
            pltpu.VMEM((idx_per_chunk, d), jnp.float32),
            pltpu.VMEM((CHUNK * d,), jnp.float32),
            pltpu.VMEM((CHUNK * d,), jnp.float32),
            pltpu.SemaphoreType.DMA,
            pltpu.SemaphoreType.DMA,
            pltpu.SemaphoreType.DMA,
            pltpu.SemaphoreType.DMA,
            pltpu.SemaphoreType.DMA,
            pltpu.SemaphoreType.DMA,
        ],
        compiler_params=pltpu.CompilerParams(
            use_tc_tiling_on_sc=False, needs_layout_passes=False),
    )
    def body(w_hbm, sidx_hbm, out_hbm,
             lidx0, lidx1, gidx0, gidx1, rows0, rows1, outc0, outc1,
             si0, si1, sg0, sg1, so0, so1):
        lidx = (lidx0, lidx1)
        gidx = (gidx0, gidx1)
        rows = (rows0, rows1)
        outc = (outc0, outc1)
        sem_i = (si0, si1)
        sem_g = (sg0, sg1)
        sem_o = (so0, so1)

        wid = lax.axis_index("s") * num_cores + lax.axis_index("c")
        c0 = wid * n
        lane128 = lax.iota(jnp.int32, LANES) * STREAM

        def idx_desc(c, s):
            return pltpu.make_async_copy(
                sidx_hbm.at[pl.ds((c0 + c) * idx_per_chunk, idx_per_chunk)],
                lidx[s], sem_i[s])

        def gather_descs(s):
            return [
                pltpu.make_async_copy(
                    w_hbm.at[gidx[s].at[pl.ds(j * STREAM, STREAM)]],
                    rows[s].at[pl.ds(j * STREAM, STREAM)],
                    sem_g[s])
                for j in range(n_streams)
            ]

        def out_descs(c, s):
            tc = c0 + c
            return [
                pltpu.make_async_copy(
                    outc[s].at[pl.ds(tr * tile_elems, tile_elems)],
                    out_hbm.at[pl.ds((tr * n_chunks + tc) * tile_elems,
                                     tile_elems)],
                    sem_o[s])
                for tr in range(d // 8)
            ]

        def adjust(c, s):
            table = (c0 + c) // chunks_per_table
            off = jnp.full((LANES,), 0, jnp.int32) + table * rows_per_table

            @plsc.parallel_loop(0, idx_per_chunk // LANES, unroll=4)
            def _(v):
                sl = pl.ds(v * LANES, LANES)
                gidx[s][sl] = lidx[s][sl] + off

        def accumulate(s):
            @plsc.parallel_loop(0, CHUNK, unroll=2)
            def _(b):
                base = b * pool
                a = rows[s][base, :]
                for j in range(1, pool):
                    a = a + rows[s][base + j, :]
                plsc.store_scatter(outc[s], [lane128 + b], a)

        def front(c, s, g):
            idx_desc(c, s).wait()
            adjust(c, s)
            for dsc in gather_descs(s):
                dsc.start()

            @pl.when(g < pairs - 1)
            def _():
                idx_desc(c + 2, s).start()

        def back(c, s):
            @pl.when(c >= 2)
            def _():
                for dsc in out_descs(c - 2, s):
                    dsc.wait()
            for dsc in gather_descs(s):
                dsc.wait()
            accumulate(s)
            for dsc in out_descs(c, s):
                dsc.start()

        idx_desc(0, 0).start()
        idx_desc(1, 1).start()

        def pair_body(g, carry):
            c = 2 * g
            front(c, 0, g)

            @pl.when(g > 0)
            def _():
                back(c - 1, 1)
            front(c + 1, 1, g)
            back(c, 0)
            return carry

        lax.fori_loop(0, pairs, pair_body, 0)
        back(n - 1, 1)
        for dsc in out_descs(n - 2, 0):
            dsc.wait()
        for dsc in out_descs(n - 1, 1):
            dsc.wait()

    return body


def kernel(weights, table_offsets, sparse_indices, sparse_offsets):
    n_bags = sparse_offsets.shape[0] - 1
    n_tables = table_offsets.shape[0]
    batch = n_bags // n_tables
    pool = sparse_indices.shape[0] // n_bags
    n_rows, d = weights.shape

    info = plsc.get_sparse_core_info()
    num_cores = info.num_cores
    nw = num_cores * info.num_subcores

    n_full = n_rows // W
    tail_arr = weights[n_full * W:, :].reshape((n_rows - n_full * W) * d)

    conv = _make_conv(n_rows, d, num_cores, nw)
    wlin = conv(weights.T, tail_arr)
    w2 = wlin.reshape(n_rows, d)

    bag = _make_bag(n_rows, d, n_bags, n_tables, batch, pool, num_cores, nw)
    flat = bag(w2, sparse_indices)
    a = flat.reshape(d // 8, n_bags // STREAM, 8, STREAM)
    return a.transpose(1, 3, 0, 2).reshape(n_bags, d)
